# baseline (speedup 1.0000x reference)
import functools

import jax
import jax.numpy as jnp
from jax.experimental import pallas as pl
from jax.experimental.pallas import tpu as pltpu


def _probe_block(p_ref, o_ref, *, block_rows, block_cols, row_stride):
    i = pl.program_id(0)
    j = pl.program_id(1)
    shape = p_ref.shape
    row = jax.lax.broadcasted_iota(jnp.uint32, shape, 0)
    col = jax.lax.broadcasted_iota(jnp.uint32, shape, 1)
    base = (
        jnp.uint32(block_rows) * jnp.uint32(i) * jnp.uint32(row_stride)
        + jnp.uint32(block_cols) * jnp.uint32(j)
        + jnp.uint32(42)
    )
    x1 = row * jnp.uint32(row_stride) + col + base
    x0 = x1
    for r in (13, 15, 26, 6, 17):
        x0 = x0 + x1
        x1 = ((x1 << jnp.uint32(r)) | (x1 >> jnp.uint32(32 - r))) ^ x0
    bits = x0 ^ x1
    fbits = (bits >> jnp.uint32(9)) | jnp.uint32(0x3F800000)
    u = jax.lax.bitcast_convert_type(fbits, jnp.float32) - jnp.float32(1.0)
    o_ref[...] = (u < p_ref[...]).astype(jnp.float32)


@jax.jit
def kernel(input):
    rows, cols = input.shape
    block_rows = 256
    block_cols = 2048
    grid = (pl.cdiv(rows, block_rows), pl.cdiv(cols, block_cols))
    return pl.pallas_call(
        functools.partial(
            _probe_block,
            block_rows=block_rows,
            block_cols=block_cols,
            row_stride=cols,
        ),
        grid=grid,
        in_specs=[pl.BlockSpec((block_rows, block_cols), lambda i, j: (i, j))],
        out_specs=pl.BlockSpec((block_rows, block_cols), lambda i, j: (i, j)),
        out_shape=jax.ShapeDtypeStruct((rows, cols), jnp.float32),
        compiler_params=pltpu.CompilerParams(
            dimension_semantics=("parallel", "parallel"),
        ),
    )(input)
